# Initial kernel scaffold; baseline (speedup 1.0000x reference)
import jax
import jax.numpy as jnp
from jax.experimental import pallas as pl


def kernel(i, j, k, labels, ue0, ie0, te0, ue1, ie1, te1, ue2, ie2, te2, ue3, ie3, te3):
    raise NotImplementedError("write your pallas kernel here")


# speedup vs baseline: 1.7685x; 1.7685x over previous
import functools

import jax
import jax.numpy as jnp
from jax import lax
from jax.experimental import pallas as pl
from jax.experimental.pallas import tpu as pltpu
from jax.experimental.pallas import tpu_sc as plsc

B = 16384
D = 128
_info = plsc.get_sparse_core_info()
NC, NS, L = _info.num_cores, _info.num_subcores, _info.num_lanes
NW = NC * NS
PW = B // NW
NCH = PW // L
CAP = PW + L

assert B % (8 * NW) == 0 and PW % L == 0

_mesh = plsc.VectorSubcoreMesh(core_axis_name="c", subcore_axis_name="s")


@functools.partial(
    pl.kernel,
    out_type=jax.ShapeDtypeStruct((B,), jnp.float32),
    mesh=_mesh,
    scratch_types=[
        pltpu.VMEM((PW,), jnp.int32),
        pltpu.VMEM((PW,), jnp.int32),
        pltpu.VMEM((PW,), jnp.int32),
        pltpu.VMEM((PW,), jnp.int32),
        *[pltpu.VMEM((CAP,), jnp.int32) for _ in range(16)],
        pltpu.VMEM((L, D), jnp.float32),
        pltpu.VMEM((L, D), jnp.float32),
        pltpu.VMEM((L, D), jnp.float32),
        pltpu.VMEM((PW,), jnp.float32),
        pltpu.SemaphoreType.DMA,
        pltpu.SemaphoreType.DMA,
        pltpu.SemaphoreType.DMA,
    ],
    compiler_params=pltpu.CompilerParams(needs_layout_passes=False),
)
def _mergl_sc(i_h, j_h, k_h, lab_h,
              ue0, ie0, te0, ue1, ie1, te1, ue2, ie2, te2, ue3, ie3, te3,
              out_h,
              iv, jv, kv, lv,
              ci0, ci1, ci2, ci3, cj0, cj1, cj2, cj3,
              ck0, ck1, ck2, ck3, cp0, cp1, cp2, cp3,
              ub, vb, tb, ob, s0, s1, s2):
    ci = (ci0, ci1, ci2, ci3)
    cj = (cj0, cj1, cj2, cj3)
    ck = (ck0, ck1, ck2, ck3)
    cp = (cp0, cp1, cp2, cp3)
    wid = lax.axis_index("s") * NC + lax.axis_index("c")
    base = wid * PW
    lane = lax.iota(jnp.int32, L)
    ones = jnp.ones((L,), jnp.int32)
    zeros = jnp.zeros((L,), jnp.int32)

    pltpu.sync_copy(i_h.at[pl.ds(base, PW)], iv)
    pltpu.sync_copy(j_h.at[pl.ds(base, PW)], jv)
    pltpu.sync_copy(k_h.at[pl.ds(base, PW)], kv)
    pltpu.sync_copy(lab_h.at[pl.ds(base, PW)], lv)

    def comp_body(c, cnts):
        off = c * L
        l16 = lv[pl.ds(off, L)]
        i16 = iv[pl.ds(off, L)]
        j16 = jv[pl.ds(off, L)]
        k16 = kv[pl.ds(off, L)]
        p16 = off + lane
        new = []
        for lbl in range(4):
            m = l16 == lbl
            mi = jnp.where(m, ones, zeros)
            dest = cnts[lbl] + plsc.cumsum(mi) - mi
            plsc.store_scatter(ci[lbl], [dest], i16, mask=m)
            plsc.store_scatter(cj[lbl], [dest], j16, mask=m)
            plsc.store_scatter(ck[lbl], [dest], k16, mask=m)
            plsc.store_scatter(cp[lbl], [dest], p16, mask=m)
            new.append(cnts[lbl] + jnp.sum(mi))
        return tuple(new)

    zero = jnp.int32(0)
    cnts = lax.fori_loop(0, NCH, comp_body, (zero, zero, zero, zero))

    tables = ((ue0, ie0, te0), (ue1, ie1, te1),
              (ue2, ie2, te2), (ue3, ie3, te3))
    for lbl in range(4):
        ue, ie, te = tables[lbl]
        nl = cnts[lbl]

        def chunk_body(off, nl=nl, ue=ue, ie=ie, te=te, lbl=lbl):
            ok = lane < (nl - off)
            gi = jnp.where(ok, ci[lbl][pl.ds(off, L)], zeros)
            gj = jnp.where(ok, cj[lbl][pl.ds(off, L)], zeros)
            gk = jnp.where(ok, ck[lbl][pl.ds(off, L)], zeros)
            cu = pltpu.async_copy(ue.at[gi], ub, s0)
            cv = pltpu.async_copy(ie.at[gj], vb, s1)
            ct = pltpu.async_copy(te.at[gk], tb, s2)
            cu.wait()
            cv.wait()
            ct.wait()

            def dot_body(q, accs):
                a0, a1 = accs
                for t in range(4):
                    col = jnp.broadcast_to(q * 4 + t, (L,)).astype(jnp.int32)
                    u = plsc.load_gather(ub, [lane, col])
                    v = plsc.load_gather(vb, [lane, col])
                    w = plsc.load_gather(tb, [lane, col])
                    if t % 2 == 0:
                        a0 = a0 + u * v * w
                    else:
                        a1 = a1 + u * v * w
                return (a0, a1)

            zf = jnp.zeros((L,), jnp.float32)
            a0, a1 = lax.fori_loop(0, D // 4, dot_body, (zf, zf))
            p16 = cp[lbl][pl.ds(off, L)]
            plsc.store_scatter(ob, [p16], a0 + a1, mask=ok)
            return off + L

        lax.while_loop(lambda off, nl=nl: off < nl, chunk_body, zero)

    pltpu.sync_copy(ob, out_h.at[pl.ds(base, PW)])


def kernel(i, j, k, labels,
           ue0, ie0, te0, ue1, ie1, te1, ue2, ie2, te2, ue3, ie3, te3):
    i = i.astype(jnp.int32)
    j = j.astype(jnp.int32)
    k = k.astype(jnp.int32)
    labels = labels.astype(jnp.int32)
    return _mergl_sc(i, j, k, labels,
                     ue0, ie0, te0, ue1, ie1, te1,
                     ue2, ie2, te2, ue3, ie3, te3)
